# baseline (speedup 1.0000x reference)
import functools

import jax
import jax.numpy as jnp
from jax import lax
from jax.experimental import pallas as pl
from jax.experimental.pallas import tpu as pltpu
from jax.experimental.pallas import tpu_sc as plsc

SEQ_LEN, BATCH, DIM = 200, 4096, 64
NUM_EMB = 1000000
TOTAL = SEQ_LEN * BATCH
NC, NS = 2, 16
NW = NC * NS
CHUNK = 256
NBLOCK = TOTAL // CHUNK
B_PER_W = NBLOCK // NW
NPAIR = B_PER_W // 2
SUB = CHUNK // 128

_mesh = plsc.VectorSubcoreMesh(core_axis_name="c", subcore_axis_name="s")


@functools.partial(
    pl.kernel,
    out_type=jax.ShapeDtypeStruct((SEQ_LEN * DIM * BATCH // 128, 128),
                                  jnp.float32),
    mesh=_mesh,
    scratch_types=[
        pltpu.VMEM((CHUNK,), jnp.int32),
        pltpu.VMEM((CHUNK,), jnp.int32),
        pltpu.VMEM((CHUNK, 8, 8), jnp.float32),
        pltpu.VMEM((CHUNK, 8, 8), jnp.float32),
        pltpu.VMEM((DIM // 8, 8, CHUNK), jnp.float32),
        pltpu.VMEM((DIM // 8, 8, CHUNK), jnp.float32),
        pltpu.SemaphoreType.DMA,
        pltpu.SemaphoreType.DMA,
        pltpu.SemaphoreType.DMA,
        pltpu.SemaphoreType.DMA,
    ],
    compiler_params=pltpu.CompilerParams(use_tc_tiling_on_sc=False,
                                         needs_layout_passes=False),
)
def _gather_kernel(idx_hbm, table_hbm, out_hbm,
                   idx0, idx1, rows0, rows1, t0, t1, sg0, sg1, so0, so1):
    wid = lax.axis_index("s") * NC + lax.axis_index("c")
    bbase = wid * B_PER_W

    def fetch(b, idx_v, rows_v, sg):
        off = pl.multiple_of(b * CHUNK, 8)
        pltpu.sync_copy(idx_hbm.at[pl.ds(off, CHUNK)], idx_v)
        pltpu.make_async_copy(table_hbm.at[idx_v], rows_v, sg).start()

    def wait_gather(idx_v, rows_v, sg):
        pltpu.make_async_copy(table_hbm.at[idx_v], rows_v, sg).wait()

    def transpose(rows_v, t_v):
        lanes = lax.iota(jnp.int32, 16)

        @plsc.parallel_loop(0, CHUNK // 16, unroll=2)
        def jbody(jb):
            jvec = jb * 16 + lanes
            for f0 in range(0, DIM, 16):
                for k in range(16):
                    fv = f0 + ((lanes + k) & 15)
                    ftv = fv >> 3
                    flv = fv & 7
                    vals = plsc.load_gather(rows_v, [jvec, ftv, flv])
                    plsc.store_scatter(t_v, [ftv, flv, jvec], vals)

    def wb(b, t_v, so, start):
        tmp = b // 4
        rq = b - tmp * 4
        tc = tmp % 32
        tr = tmp // 32
        for rr in range(SUB):
            s = tr * 8 + rq * 2 + rr
            for ft in range(DIM // 8):
                src = t_v.at[ft, :, pl.ds(rr * 128, 128)]
                row0 = ((s * 8 + ft) * 32 + tc) * 8
                dst = out_hbm.at[pl.ds(pl.multiple_of(row0, 8), 8), :]
                cp = pltpu.make_async_copy(src, dst, so)
                if start:
                    cp.start()
                else:
                    cp.wait()

    fetch(bbase, idx0, rows0, sg0)
    fetch(bbase + 1, idx1, rows1, sg1)
    wait_gather(idx0, rows0, sg0)
    transpose(rows0, t0)
    wb(bbase, t0, so0, True)
    wait_gather(idx1, rows1, sg1)
    transpose(rows1, t1)
    wb(bbase + 1, t1, so1, True)

    def body(i, carry):
        b0 = bbase + i * 2
        b1 = b0 + 1
        fetch(b0, idx0, rows0, sg0)
        fetch(b1, idx1, rows1, sg1)
        wait_gather(idx0, rows0, sg0)
        wb(b0 - 2, t0, so0, False)
        transpose(rows0, t0)
        wb(b0, t0, so0, True)
        wait_gather(idx1, rows1, sg1)
        wb(b1 - 2, t1, so1, False)
        transpose(rows1, t1)
        wb(b1, t1, so1, True)
        return carry

    lax.fori_loop(1, NPAIR, body, 0)
    wb(bbase + B_PER_W - 2, t0, so0, False)
    wb(bbase + B_PER_W - 1, t1, so1, False)


def kernel(input, weight):
    idx = (input.astype(jnp.int32).reshape(25, 8, 32, 128)
           .transpose(0, 2, 1, 3).reshape(-1))
    out = _gather_kernel(idx, weight.reshape(NUM_EMB, 8, 8))
    out5 = out.reshape(SEQ_LEN, DIM // 8, BATCH // 128, 8, 128)
    return out5.transpose(0, 2, 4, 1, 3).reshape(SEQ_LEN, BATCH, DIM)


# speedup vs baseline: 3.6344x; 3.6344x over previous
import functools

import jax
import jax.numpy as jnp
from jax import lax
from jax.experimental import pallas as pl
from jax.experimental.pallas import tpu as pltpu
from jax.experimental.pallas import tpu_sc as plsc

SEQ_LEN, BATCH, DIM = 200, 4096, 64
TOTAL = SEQ_LEN * BATCH
NC, NS = 2, 16
NW = NC * NS
CHUNK = 256
NBLOCK = TOTAL // CHUNK
B_PER_W = NBLOCK // NW
NPAIR = B_PER_W // 2
SUB = CHUNK // 128

_mesh = plsc.VectorSubcoreMesh(core_axis_name="c", subcore_axis_name="s")


@functools.partial(
    pl.kernel,
    out_type=jax.ShapeDtypeStruct((SEQ_LEN, DIM // 8, BATCH // 128, 8, 128),
                                  jnp.float32),
    mesh=_mesh,
    scratch_types=[
        pltpu.VMEM((CHUNK,), jnp.int32),
        pltpu.VMEM((CHUNK,), jnp.int32),
        pltpu.VMEM((CHUNK, DIM), jnp.float32),
        pltpu.VMEM((CHUNK, DIM), jnp.float32),
        pltpu.VMEM((DIM // 8, 8, CHUNK), jnp.float32),
        pltpu.VMEM((DIM // 8, 8, CHUNK), jnp.float32),
        pltpu.SemaphoreType.DMA,
        pltpu.SemaphoreType.DMA,
        pltpu.SemaphoreType.DMA,
        pltpu.SemaphoreType.DMA,
    ],
    compiler_params=pltpu.CompilerParams(use_tc_tiling_on_sc=False,
                                         needs_layout_passes=False),
)
def _gather_kernel(idx_hbm, table_hbm, out_hbm,
                   idx0, idx1, rows0, rows1, t0, t1, sg0, sg1, so0, so1):
    wid = lax.axis_index("s") * NC + lax.axis_index("c")
    bbase = wid * B_PER_W

    def fetch(b, idx_v, rows_v, sg):
        off = pl.multiple_of(b * CHUNK, 8)
        pltpu.sync_copy(idx_hbm.at[pl.ds(off, CHUNK)], idx_v)
        pltpu.make_async_copy(table_hbm.at[idx_v], rows_v, sg).start()

    def wait_gather(idx_v, rows_v, sg):
        pltpu.make_async_copy(table_hbm.at[idx_v], rows_v, sg).wait()

    def transpose(rows_v, t_v):
        def jbody(jb, carry):
            lanes = lax.iota(jnp.int32, 16)
            jvec = jb * 16 + lanes
            for f0 in range(0, DIM, 16):
                for kg in range(2):
                    fvs = [f0 + ((lanes + kg * 8 + k) & 15) for k in range(8)]
                    vals = [plsc.load_gather(rows_v, [jvec, fv]) for fv in fvs]
                    for fv, v in zip(fvs, vals):
                        plsc.store_scatter(t_v, [fv >> 3, fv & 7, jvec], v)
            return carry

        lax.fori_loop(0, CHUNK // 16, jbody, 0)

    def wb(b, t_v, so, start):
        tmp = b // 4
        rq = b - tmp * 4
        tc = tmp % 32
        tr = tmp // 32
        for rr in range(SUB):
            s = tr * 8 + rq * 2 + rr
            src = t_v.at[:, :, pl.ds(rr * 128, 128)]
            dst = out_hbm.at[s, :, tc, :, :]
            cp = pltpu.make_async_copy(src, dst, so)
            if start:
                cp.start()
            else:
                cp.wait()

    fetch(bbase, idx0, rows0, sg0)
    fetch(bbase + 1, idx1, rows1, sg1)
    wait_gather(idx0, rows0, sg0)
    transpose(rows0, t0)
    wb(bbase, t0, so0, True)
    wait_gather(idx1, rows1, sg1)
    transpose(rows1, t1)
    wb(bbase + 1, t1, so1, True)

    def body(i, carry):
        b0 = bbase + i * 2
        b1 = b0 + 1
        fetch(b0, idx0, rows0, sg0)
        fetch(b1, idx1, rows1, sg1)
        wait_gather(idx0, rows0, sg0)
        wb(b0 - 2, t0, so0, False)
        transpose(rows0, t0)
        wb(b0, t0, so0, True)
        wait_gather(idx1, rows1, sg1)
        wb(b1 - 2, t1, so1, False)
        transpose(rows1, t1)
        wb(b1, t1, so1, True)
        return carry

    lax.fori_loop(1, NPAIR, body, 0)
    wb(bbase + B_PER_W - 2, t0, so0, False)
    wb(bbase + B_PER_W - 1, t1, so1, False)


def kernel(input, weight):
    idx = (input.astype(jnp.int32).reshape(25, 8, 32, 128)
           .transpose(0, 2, 1, 3).reshape(-1))
    out = _gather_kernel(idx, weight)
    return out.transpose(0, 2, 4, 1, 3).reshape(SEQ_LEN, BATCH, DIM)


# speedup vs baseline: 3.7463x; 1.0308x over previous
import functools

import jax
import jax.numpy as jnp
from jax import lax
from jax.experimental import pallas as pl
from jax.experimental.pallas import tpu as pltpu
from jax.experimental.pallas import tpu_sc as plsc

SEQ_LEN, BATCH, DIM = 200, 4096, 64
TOTAL = SEQ_LEN * BATCH
NC, NS = 2, 16
NW = NC * NS
CHUNK = 256
NBLOCK = TOTAL // CHUNK
B_PER_W = NBLOCK // NW
NPAIR = B_PER_W // 2
SUB = CHUNK // 128

_mesh = plsc.VectorSubcoreMesh(core_axis_name="c", subcore_axis_name="s")


@functools.partial(
    pl.kernel,
    out_type=jax.ShapeDtypeStruct((SEQ_LEN, DIM // 8, BATCH // 128, 8, 128),
                                  jnp.float32),
    mesh=_mesh,
    scratch_types=[
        pltpu.VMEM((CHUNK,), jnp.int32),
        pltpu.VMEM((CHUNK,), jnp.int32),
        pltpu.VMEM((CHUNK, DIM), jnp.float32),
        pltpu.VMEM((CHUNK, DIM), jnp.float32),
        pltpu.VMEM((DIM // 8, 8, CHUNK), jnp.float32),
        pltpu.VMEM((DIM // 8, 8, CHUNK), jnp.float32),
        pltpu.SemaphoreType.DMA,
        pltpu.SemaphoreType.DMA,
        pltpu.SemaphoreType.DMA,
        pltpu.SemaphoreType.DMA,
    ],
    compiler_params=pltpu.CompilerParams(use_tc_tiling_on_sc=False,
                                         needs_layout_passes=False),
)
def _gather_kernel(idx_hbm, table_hbm, out_hbm,
                   idx0, idx1, rows0, rows1, t0, t1, sg0, sg1, so0, so1):
    wid = lax.axis_index("s") * NC + lax.axis_index("c")
    bbase = wid * B_PER_W

    def fetch(b, idx_v, rows_v, sg):
        off = pl.multiple_of(b * CHUNK, 8)
        pltpu.sync_copy(idx_hbm.at[pl.ds(off, CHUNK)], idx_v)
        pltpu.make_async_copy(table_hbm.at[idx_v], rows_v, sg).start()

    def wait_gather(idx_v, rows_v, sg):
        pltpu.make_async_copy(table_hbm.at[idx_v], rows_v, sg).wait()

    def transpose(rows_v, t_v):
        def jbody(jb, carry):
            lanes = lax.iota(jnp.int32, 16)
            jvec = jb * 16 + lanes
            for f0 in range(0, DIM, 16):
                fvs = [f0 + ((lanes + k) & 15) for k in range(16)]
                vals = [plsc.load_gather(rows_v, [jvec, fv]) for fv in fvs]
                for fv, v in zip(fvs, vals):
                    plsc.store_scatter(t_v, [fv >> 3, fv & 7, jvec], v)
            return carry

        lax.fori_loop(0, CHUNK // 16, jbody, 0)

    def wb(b, t_v, so, start):
        tmp = b // 4
        rq = b - tmp * 4
        tc = tmp % 32
        tr = tmp // 32
        for rr in range(SUB):
            s = tr * 8 + rq * 2 + rr
            src = t_v.at[:, :, pl.ds(rr * 128, 128)]
            dst = out_hbm.at[s, :, tc, :, :]
            cp = pltpu.make_async_copy(src, dst, so)
            if start:
                cp.start()
            else:
                cp.wait()

    fetch(bbase, idx0, rows0, sg0)
    fetch(bbase + 1, idx1, rows1, sg1)
    wait_gather(idx0, rows0, sg0)
    transpose(rows0, t0)
    wb(bbase, t0, so0, True)
    wait_gather(idx1, rows1, sg1)
    transpose(rows1, t1)
    wb(bbase + 1, t1, so1, True)

    def body(i, carry):
        b0 = bbase + i * 2
        b1 = b0 + 1
        fetch(b0, idx0, rows0, sg0)
        fetch(b1, idx1, rows1, sg1)
        wait_gather(idx0, rows0, sg0)
        wb(b0 - 2, t0, so0, False)
        transpose(rows0, t0)
        wb(b0, t0, so0, True)
        wait_gather(idx1, rows1, sg1)
        wb(b1 - 2, t1, so1, False)
        transpose(rows1, t1)
        wb(b1, t1, so1, True)
        return carry

    lax.fori_loop(1, NPAIR, body, 0)
    wb(bbase + B_PER_W - 2, t0, so0, False)
    wb(bbase + B_PER_W - 1, t1, so1, False)


def kernel(input, weight):
    idx = (input.astype(jnp.int32).reshape(25, 8, 32, 128)
           .transpose(0, 2, 1, 3).reshape(-1))
    out = _gather_kernel(idx, weight)
    return out.transpose(0, 2, 4, 1, 3).reshape(SEQ_LEN, BATCH, DIM)


# speedup vs baseline: 3.9252x; 1.0477x over previous
import functools

import jax
import jax.numpy as jnp
from jax import lax
from jax.experimental import pallas as pl
from jax.experimental.pallas import tpu as pltpu
from jax.experimental.pallas import tpu_sc as plsc

SEQ_LEN, BATCH, DIM = 200, 4096, 64
TOTAL = SEQ_LEN * BATCH
NC, NS = 2, 16
NW = NC * NS
CHUNK = 256
NBLOCK = TOTAL // CHUNK
B_PER_W = NBLOCK // NW
NPAIR = B_PER_W // 2
SUB = CHUNK // 128

_mesh = plsc.VectorSubcoreMesh(core_axis_name="c", subcore_axis_name="s")


@functools.partial(
    pl.kernel,
    out_type=jax.ShapeDtypeStruct((SEQ_LEN, DIM // 8, BATCH // 128, 8, 128),
                                  jnp.float32),
    mesh=_mesh,
    scratch_types=[
        pltpu.VMEM((CHUNK,), jnp.int32),
        pltpu.VMEM((CHUNK,), jnp.int32),
        pltpu.VMEM((CHUNK, DIM), jnp.float32),
        pltpu.VMEM((CHUNK, DIM), jnp.float32),
        pltpu.VMEM((DIM // 8, 8, CHUNK), jnp.float32),
        pltpu.VMEM((DIM // 8, 8, CHUNK), jnp.float32),
        pltpu.SemaphoreType.DMA,
        pltpu.SemaphoreType.DMA,
        pltpu.SemaphoreType.DMA,
        pltpu.SemaphoreType.DMA,
    ],
    compiler_params=pltpu.CompilerParams(use_tc_tiling_on_sc=False,
                                         needs_layout_passes=False),
)
def _gather_kernel(idx_hbm, table_hbm, out_hbm,
                   idx0, idx1, rows0, rows1, t0, t1, sg0, sg1, so0, so1):
    wid = lax.axis_index("s") * NC + lax.axis_index("c")
    bbase = wid * B_PER_W

    def fetch(b, idx_v, rows_v, sg):
        off = pl.multiple_of(b * CHUNK, 8)
        pltpu.sync_copy(idx_hbm.at[pl.ds(off, CHUNK)], idx_v)
        pltpu.make_async_copy(table_hbm.at[idx_v], rows_v, sg).start()

    def wait_gather(idx_v, rows_v, sg):
        pltpu.make_async_copy(table_hbm.at[idx_v], rows_v, sg).wait()

    def transpose(rows_v, t_v):
        def jbody(jb, carry):
            lanes = lax.iota(jnp.int32, 16)
            jvec = jb * 16 + lanes
            for f0 in range(0, DIM, 16):
                fvs = [f0 + ((lanes + k) & 15) for k in range(16)]
                vals = [plsc.load_gather(rows_v, [jvec, fv]) for fv in fvs]
                for fv, v in zip(fvs, vals):
                    plsc.store_scatter(t_v, [fv >> 3, fv & 7, jvec], v)
            return carry

        lax.fori_loop(0, CHUNK // 16, jbody, 0)

    def wb(b, t_v, so, start):
        tmp = b // 4
        rq = b - tmp * 4
        tc = tmp % 32
        tr = tmp // 32
        for rr in range(SUB):
            s = tr * 8 + rq * 2 + rr
            src = t_v.at[:, :, pl.ds(rr * 128, 128)]
            dst = out_hbm.at[s, :, tc, :, :]
            cp = pltpu.make_async_copy(src, dst, so)
            if start:
                cp.start()
            else:
                cp.wait()

    fetch(bbase, idx0, rows0, sg0)
    fetch(bbase + 1, idx1, rows1, sg1)
    wait_gather(idx0, rows0, sg0)
    transpose(rows0, t0)
    wb(bbase, t0, so0, True)
    fetch(bbase + 2, idx0, rows0, sg0)
    wait_gather(idx1, rows1, sg1)
    transpose(rows1, t1)
    wb(bbase + 1, t1, so1, True)
    fetch(bbase + 3, idx1, rows1, sg1)

    def body(i, carry):
        b0 = bbase + i * 2
        b1 = b0 + 1
        wait_gather(idx0, rows0, sg0)
        wb(b0 - 2, t0, so0, False)
        transpose(rows0, t0)
        wb(b0, t0, so0, True)
        fetch(b0 + 2, idx0, rows0, sg0)
        wait_gather(idx1, rows1, sg1)
        wb(b1 - 2, t1, so1, False)
        transpose(rows1, t1)
        wb(b1, t1, so1, True)
        fetch(b1 + 2, idx1, rows1, sg1)
        return carry

    lax.fori_loop(1, NPAIR - 1, body, 0)
    blast0 = bbase + B_PER_W - 2
    blast1 = bbase + B_PER_W - 1
    wait_gather(idx0, rows0, sg0)
    wb(blast0 - 2, t0, so0, False)
    transpose(rows0, t0)
    wb(blast0, t0, so0, True)
    wait_gather(idx1, rows1, sg1)
    wb(blast1 - 2, t1, so1, False)
    transpose(rows1, t1)
    wb(blast1, t1, so1, True)
    wb(blast0, t0, so0, False)
    wb(blast1, t1, so1, False)


def kernel(input, weight):
    idx = (input.astype(jnp.int32).reshape(25, 8, 32, 128)
           .transpose(0, 2, 1, 3).reshape(-1))
    out = _gather_kernel(idx, weight)
    return out.transpose(0, 2, 4, 1, 3).reshape(SEQ_LEN, BATCH, DIM)


# speedup vs baseline: 4.0236x; 1.0251x over previous
import functools

import jax
import jax.numpy as jnp
from jax import lax
from jax.experimental import pallas as pl
from jax.experimental.pallas import tpu as pltpu
from jax.experimental.pallas import tpu_sc as plsc

SEQ_LEN, BATCH, DIM = 200, 4096, 64
TOTAL = SEQ_LEN * BATCH
NC, NS = 2, 16
NW = NC * NS
CHUNK = 256
NBLOCK = TOTAL // CHUNK
B_PER_W = NBLOCK // NW
NPAIR = B_PER_W // 2
SUB = CHUNK // 128

_mesh = plsc.VectorSubcoreMesh(core_axis_name="c", subcore_axis_name="s")


@functools.partial(
    pl.kernel,
    out_type=jax.ShapeDtypeStruct((SEQ_LEN, DIM // 8, BATCH // 128, 8, 128),
                                  jnp.float32),
    mesh=_mesh,
    scratch_types=[
        pltpu.VMEM((CHUNK,), jnp.int32),
        pltpu.VMEM((CHUNK,), jnp.int32),
        pltpu.VMEM((CHUNK, DIM), jnp.float32),
        pltpu.VMEM((CHUNK, DIM), jnp.float32),
        pltpu.VMEM((DIM // 8, 8, CHUNK), jnp.float32),
        pltpu.VMEM((DIM // 8, 8, CHUNK), jnp.float32),
        pltpu.SemaphoreType.DMA,
        pltpu.SemaphoreType.DMA,
        pltpu.SemaphoreType.DMA,
        pltpu.SemaphoreType.DMA,
    ],
    compiler_params=pltpu.CompilerParams(use_tc_tiling_on_sc=False,
                                         needs_layout_passes=False),
)
def _gather_kernel(idx_hbm, table_hbm, out_hbm,
                   idx0, idx1, rows0, rows1, t0, t1, sg0, sg1, so0, so1):
    wid = lax.axis_index("s") * NC + lax.axis_index("c")
    bbase = wid * B_PER_W

    def fetch(b, idx_v, rows_v, sg):
        off = pl.multiple_of(b * CHUNK, 8)
        pltpu.sync_copy(idx_hbm.at[pl.ds(off, CHUNK)], idx_v)
        pltpu.make_async_copy(table_hbm.at[idx_v], rows_v, sg).start()

    def wait_gather(idx_v, rows_v, sg):
        pltpu.make_async_copy(table_hbm.at[idx_v], rows_v, sg).wait()

    def transpose(rows_v, t_v):
        def jbody(jb, carry):
            lanes = lax.iota(jnp.int32, 16)
            jvecs = [(jb * 2 + h) * 16 + lanes for h in range(2)]
            for f0 in range(0, DIM, 16):
                fvs = [f0 + ((lanes + k) & 15) for k in range(16)]
                for jvec in jvecs:
                    vals = [plsc.load_gather(rows_v, [jvec, fv]) for fv in fvs]
                    for fv, v in zip(fvs, vals):
                        plsc.store_scatter(t_v, [fv >> 3, fv & 7, jvec], v)
            return carry

        lax.fori_loop(0, CHUNK // 32, jbody, 0)

    def wb(b, t_v, so, start):
        tmp = b // 4
        rq = b - tmp * 4
        tc = tmp % 32
        tr = tmp // 32
        for rr in range(SUB):
            s = tr * 8 + rq * 2 + rr
            src = t_v.at[:, :, pl.ds(rr * 128, 128)]
            dst = out_hbm.at[s, :, tc, :, :]
            cp = pltpu.make_async_copy(src, dst, so)
            if start:
                cp.start()
            else:
                cp.wait()

    fetch(bbase, idx0, rows0, sg0)
    fetch(bbase + 1, idx1, rows1, sg1)
    wait_gather(idx0, rows0, sg0)
    transpose(rows0, t0)
    wb(bbase, t0, so0, True)
    fetch(bbase + 2, idx0, rows0, sg0)
    wait_gather(idx1, rows1, sg1)
    transpose(rows1, t1)
    wb(bbase + 1, t1, so1, True)
    fetch(bbase + 3, idx1, rows1, sg1)

    def body(i, carry):
        b0 = bbase + i * 2
        b1 = b0 + 1
        wait_gather(idx0, rows0, sg0)
        wb(b0 - 2, t0, so0, False)
        transpose(rows0, t0)
        wb(b0, t0, so0, True)
        fetch(b0 + 2, idx0, rows0, sg0)
        wait_gather(idx1, rows1, sg1)
        wb(b1 - 2, t1, so1, False)
        transpose(rows1, t1)
        wb(b1, t1, so1, True)
        fetch(b1 + 2, idx1, rows1, sg1)
        return carry

    lax.fori_loop(1, NPAIR - 1, body, 0)
    blast0 = bbase + B_PER_W - 2
    blast1 = bbase + B_PER_W - 1
    wait_gather(idx0, rows0, sg0)
    wb(blast0 - 2, t0, so0, False)
    transpose(rows0, t0)
    wb(blast0, t0, so0, True)
    wait_gather(idx1, rows1, sg1)
    wb(blast1 - 2, t1, so1, False)
    transpose(rows1, t1)
    wb(blast1, t1, so1, True)
    wb(blast0, t0, so0, False)
    wb(blast1, t1, so1, False)


def kernel(input, weight):
    idx = (input.astype(jnp.int32).reshape(25, 8, 32, 128)
           .transpose(0, 2, 1, 3).reshape(-1))
    out = _gather_kernel(idx, weight)
    return out.transpose(0, 2, 4, 1, 3).reshape(SEQ_LEN, BATCH, DIM)
